# baseline (speedup 1.0000x reference)
import functools

import jax
import jax.numpy as jnp
from jax import lax
from jax.experimental import pallas as pl
from jax.experimental.pallas import tpu as pltpu
from jax.experimental.pallas import tpu_sc as plsc

BATCH = 16384
EMBED_DIM = 64
NUM_WORKERS = 32
ROWS_PER_W = BATCH // NUM_WORKERS
IDX_MINOR = 128
IDX_ROWS = ROWS_PER_W // IDX_MINOR
GROUPS = ROWS_PER_W // 16


def _sc_body(uids, iids, uemb, iemb, ubias, ibias, gbias, out,
             uidx_v, iidx_v, urows_v, irows_v, ubias_v, ibias_v,
             gb_v, scores_v, scratch_v, sems, out_sem):
    nc = 2
    wid = lax.axis_index("s") * nc + lax.axis_index("c")
    base = wid * ROWS_PER_W

    pltpu.sync_copy(uids.at[pl.ds(wid * IDX_ROWS, IDX_ROWS)], uidx_v)
    pltpu.sync_copy(iids.at[pl.ds(wid * IDX_ROWS, IDX_ROWS)], iidx_v)
    pltpu.sync_copy(gbias, gb_v)

    chunk_copies = []
    for j in range(IDX_ROWS):
        sl = pl.ds(j * IDX_MINOR, IDX_MINOR)
        sem = sems.at[j]
        chunk_copies.append([
            pltpu.async_copy(uemb.at[uidx_v.at[j]], urows_v.at[sl], sem),
            pltpu.async_copy(iemb.at[iidx_v.at[j]], irows_v.at[sl], sem),
            pltpu.async_copy(ubias.at[uidx_v.at[j]], ubias_v.at[sl], sem),
            pltpu.async_copy(ibias.at[iidx_v.at[j]], ibias_v.at[sl], sem),
        ])

    lanes = lax.iota(jnp.int32, 16)
    gb = gb_v[:]
    lanes16 = lanes * 16

    def group_body(g, carry):
        row0 = g * 16
        for r in range(16):
            row = row0 + r
            acc = urows_v[row, pl.ds(0, 16)] * irows_v[row, pl.ds(0, 16)]
            for c in range(1, 4):
                acc = acc + (urows_v[row, pl.ds(c * 16, 16)] *
                             irows_v[row, pl.ds(c * 16, 16)])
            scratch_v[pl.ds(r * 16, 16)] = acc
        vals = [plsc.load_gather(scratch_v, [lanes16 + c]) for c in range(16)]
        vals.append(ubias_v[pl.ds(row0, 16)] + ibias_v[pl.ds(row0, 16)] + gb)
        while len(vals) > 1:
            pairs = [a + b for a, b in zip(vals[::2], vals[1::2])]
            vals = pairs + (vals[-1:] if len(vals) % 2 else [])
        scores_v[pl.ds(row0, 16)] = vals[0]
        return carry

    groups_per_chunk = IDX_MINOR // 16
    out_copies = []
    for j in range(IDX_ROWS):
        for c in chunk_copies[j]:
            c.wait()
        lax.fori_loop(j * groups_per_chunk, (j + 1) * groups_per_chunk,
                      group_body, 0)
        out_copies.append(pltpu.async_copy(
            scores_v.at[pl.ds(j * IDX_MINOR, IDX_MINOR)],
            out.at[pl.ds(base + j * IDX_MINOR, IDX_MINOR)], out_sem))
    for c in out_copies:
        c.wait()


def kernel(user_ids, item_ids, user_emb_w, item_emb_w, user_bias_w,
           item_bias_w, global_bias):
    uids = jnp.asarray(user_ids, jnp.int32).reshape(
        NUM_WORKERS * IDX_ROWS, IDX_MINOR)
    iids = jnp.asarray(item_ids, jnp.int32).reshape(
        NUM_WORKERS * IDX_ROWS, IDX_MINOR)
    gb16 = jnp.broadcast_to(global_bias.astype(jnp.float32), (16,))
    ubias1d = user_bias_w.reshape(-1)
    ibias1d = item_bias_w.reshape(-1)

    mesh = plsc.VectorSubcoreMesh(core_axis_name="c", subcore_axis_name="s")
    run = pl.kernel(
        _sc_body,
        mesh=mesh,
        compiler_params=pltpu.CompilerParams(
            needs_layout_passes=False, use_tc_tiling_on_sc=False),
        out_type=jax.ShapeDtypeStruct((BATCH,), jnp.float32),
        scratch_types=[
            pltpu.VMEM((IDX_ROWS, IDX_MINOR), jnp.int32),
            pltpu.VMEM((IDX_ROWS, IDX_MINOR), jnp.int32),
            pltpu.VMEM((ROWS_PER_W, EMBED_DIM), jnp.float32),
            pltpu.VMEM((ROWS_PER_W, EMBED_DIM), jnp.float32),
            pltpu.VMEM((ROWS_PER_W,), jnp.float32),
            pltpu.VMEM((ROWS_PER_W,), jnp.float32),
            pltpu.VMEM((16,), jnp.float32),
            pltpu.VMEM((ROWS_PER_W,), jnp.float32),
            pltpu.VMEM((256,), jnp.float32),
            pltpu.SemaphoreType.DMA((IDX_ROWS,)),
            pltpu.SemaphoreType.DMA,
        ],
    )
    return run(uids, iids, user_emb_w, item_emb_w, ubias1d, ibias1d, gb16)


# speedup vs baseline: 1.3816x; 1.3816x over previous
import functools

import jax
import jax.numpy as jnp
from jax import lax
from jax.experimental import pallas as pl
from jax.experimental.pallas import tpu as pltpu
from jax.experimental.pallas import tpu_sc as plsc

BATCH = 16384
EMBED_DIM = 64
NUM_WORKERS = 32
ROWS_PER_W = BATCH // NUM_WORKERS
NGROUP = ROWS_PER_W // 16
NBUF = 4


def _sc_body(uids_hbm, iids_hbm, uemb, iemb, ubias, ibias, gbias, out,
             uidx_v, iidx_v, urows, irows, ubias_v, ibias_v,
             gb_v, scores_v, scratch_v, usems, isems, bias_sem):
    nc = 2
    wid = lax.axis_index("s") * nc + lax.axis_index("c")
    base = wid * ROWS_PER_W

    for j in range(4):
        pltpu.sync_copy(uids_hbm.at[pl.ds(base + j * 128, 128)],
                        uidx_v.at[j])
        pltpu.sync_copy(iids_hbm.at[pl.ds(base + j * 128, 128)],
                        iidx_v.at[j])
    pltpu.sync_copy(gbias, gb_v)

    bias_copies = []
    for j in range(4):
        sl = pl.ds(j * 128, 128)
        bias_copies.append(
            pltpu.async_copy(ubias.at[uidx_v.at[j]], ubias_v.at[sl], bias_sem))
        bias_copies.append(
            pltpu.async_copy(ibias.at[iidx_v.at[j]], ibias_v.at[sl], bias_sem))

    lanes = lax.iota(jnp.int32, 16)
    lanes16 = lanes * 16
    gb = gb_v[:]

    def fire(g, buf):
        j = g // 8
        col = (g % 8) * 16
        uvec = uidx_v[j, pl.ds(col, 16)]
        ivec = iidx_v[j, pl.ds(col, 16)]
        for r in range(16):
            pltpu.async_copy(uemb.at[pl.ds(uvec[r], 1)],
                             urows.at[buf, pl.ds(r, 1)], usems.at[buf])
            pltpu.async_copy(iemb.at[pl.ds(ivec[r], 1)],
                             irows.at[buf, pl.ds(r, 1)], isems.at[buf])

    def drain(buf):
        for r in range(16):
            pltpu.make_async_copy(uemb.at[pl.ds(0, 1)],
                                  urows.at[buf, pl.ds(r, 1)],
                                  usems.at[buf]).wait()
            pltpu.make_async_copy(iemb.at[pl.ds(0, 1)],
                                  irows.at[buf, pl.ds(r, 1)],
                                  isems.at[buf]).wait()

    def compute(g, buf):
        for r in range(16):
            acc = (urows[buf, r, pl.ds(0, 16)] *
                   irows[buf, r, pl.ds(0, 16)])
            for c in range(1, 4):
                acc = acc + (urows[buf, r, pl.ds(c * 16, 16)] *
                             irows[buf, r, pl.ds(c * 16, 16)])
            scratch_v[pl.ds(r * 16, 16)] = acc
        vals = [plsc.load_gather(scratch_v, [lanes16 + c]) for c in range(16)]
        while len(vals) > 1:
            pairs = [a + b for a, b in zip(vals[::2], vals[1::2])]
            vals = pairs + (vals[-1:] if len(vals) % 2 else [])
        scores_v[pl.ds(g * 16, 16)] = vals[0]

    for g in range(NBUF - 1):
        fire(g, g)

    def step(t, carry):
        for pos in range(NBUF):
            g = t * NBUF + pos
            gf = g + NBUF - 1

            @pl.when(gf < NGROUP)
            def _():
                fire(gf, (pos + NBUF - 1) % NBUF)

            drain(pos)
            compute(g, pos)
        return carry

    lax.fori_loop(0, NGROUP // NBUF, step, 0)

    for cp in bias_copies:
        cp.wait()
    for g in range(NGROUP):
        sl = pl.ds(g * 16, 16)
        scores_v[sl] = scores_v[sl] + ubias_v[sl] + ibias_v[sl] + gb
    pltpu.sync_copy(scores_v, out.at[pl.ds(base, ROWS_PER_W)])


def kernel(user_ids, item_ids, user_emb_w, item_emb_w, user_bias_w,
           item_bias_w, global_bias):
    uids = jnp.asarray(user_ids, jnp.int32)
    iids = jnp.asarray(item_ids, jnp.int32)
    ubias1d = user_bias_w.reshape(-1)
    ibias1d = item_bias_w.reshape(-1)
    gb16 = jnp.broadcast_to(global_bias.astype(jnp.float32), (16,))

    mesh = plsc.VectorSubcoreMesh(core_axis_name="c", subcore_axis_name="s")
    run = pl.kernel(
        _sc_body,
        mesh=mesh,
        compiler_params=pltpu.CompilerParams(
            needs_layout_passes=False, use_tc_tiling_on_sc=True),
        out_type=jax.ShapeDtypeStruct((BATCH,), jnp.float32),
        scratch_types=[
            pltpu.VMEM((4, 128), jnp.int32),
            pltpu.VMEM((4, 128), jnp.int32),
            pltpu.VMEM((NBUF, 16, EMBED_DIM), jnp.float32),
            pltpu.VMEM((NBUF, 16, EMBED_DIM), jnp.float32),
            pltpu.VMEM((ROWS_PER_W,), jnp.float32),
            pltpu.VMEM((ROWS_PER_W,), jnp.float32),
            pltpu.VMEM((16,), jnp.float32),
            pltpu.VMEM((ROWS_PER_W,), jnp.float32),
            pltpu.VMEM((256,), jnp.float32),
            pltpu.SemaphoreType.DMA((NBUF,)),
            pltpu.SemaphoreType.DMA((NBUF,)),
            pltpu.SemaphoreType.DMA,
        ],
    )
    return run(uids, iids, user_emb_w, item_emb_w, ubias1d, ibias1d, gb16)
